# baseline (speedup 1.0000x reference)
import functools

import jax
import jax.numpy as jnp
from jax import lax
from jax.experimental import pallas as pl
from jax.experimental.pallas import tpu as pltpu
from jax.experimental.pallas import tpu_sc as plsc

ROWS = 1024
VOCAB = 100000
NLAB = 100
LPAD = 112
NCHUNK = LPAD // 16
LN2 = 0.6931471805599453


def _lane_reduce(v, op, lane):
    dnums = lax.GatherDimensionNumbers(
        offset_dims=(), collapsed_slice_dims=(0,), start_index_map=(0,))
    for shift in (1, 2, 4, 8):
        idx = lax.bitwise_and(lane + shift, 15)
        rot = lax.gather(v, idx.reshape(16, 1), dnums, (1,),
                         mode=lax.GatherScatterMode.PROMISE_IN_BOUNDS)
        v = op(v, rot)
    return v


def _log_splat(sv):
    bits = lax.bitcast_convert_type(sv, jnp.int32)
    e = lax.shift_right_logical(bits, 23) - 127
    mb = lax.bitwise_or(lax.bitwise_and(bits, 0x007FFFFF), 0x3F800000)
    mf = lax.bitcast_convert_type(mb, jnp.float32)
    big = mf > 1.4142135
    mf = jnp.where(big, mf * 0.5, mf)
    e = jnp.where(big, e + 1, e)
    t = (mf - 1.0) / (mf + 1.0)
    t2 = t * t
    poly = 1.0 + t2 * (
        0.3333333333 + t2 * (0.2 + t2 * (0.1428571429 + t2 * 0.1111111111)))
    return 2.0 * t * poly + e.astype(jnp.float32) * LN2


def _make_sc_call():
    info = plsc.get_sparse_core_info()
    nw = info.num_cores * info.num_subcores
    rows_per_w = ROWS // nw
    mesh = plsc.VectorSubcoreMesh(core_axis_name="c", subcore_axis_name="s")
    tail = NLAB - 16 * (NCHUNK - 1)

    @functools.partial(
        pl.kernel,
        mesh=mesh,
        out_type=jax.ShapeDtypeStruct((ROWS, LPAD), jnp.float32),
        scratch_types=[
            pltpu.VMEM((LPAD,), jnp.int32),
            pltpu.VMEM((LPAD,), jnp.int32),
            pltpu.VMEM((LPAD,), jnp.float32),
            pltpu.VMEM((rows_per_w, LPAD), jnp.float32),
            pltpu.SemaphoreType.DMA,
        ],
    )
    def sc_call(flat_hbm, w_hbm, out_hbm,
                w_v, gidx_v, vals_v, outbuf_v, sem):
        wid = lax.axis_index("s") * info.num_cores + lax.axis_index("c")
        base = wid * rows_per_w
        lane = lax.iota(jnp.int32, 16)

        pltpu.sync_copy(w_hbm, w_v)

        def row_body(il, carry):
            off = (base + il) * VOCAB
            for c in range(NCHUNK):
                gidx_v[pl.ds(16 * c, 16)] = w_v[pl.ds(16 * c, 16)] + off
            pltpu.async_copy(flat_hbm.at[gidx_v], vals_v, sem).wait()

            maxv = jnp.full((16,), -3.4e38, jnp.float32)
            for c in range(NCHUNK):
                v = vals_v[pl.ds(16 * c, 16)]
                if c == NCHUNK - 1:
                    v = jnp.where(lane < tail, v, -3.4e38)
                maxv = jnp.maximum(maxv, v)
            m = _lane_reduce(maxv, jnp.maximum, lane)

            sumv = jnp.zeros((16,), jnp.float32)
            for c in range(NCHUNK):
                t = jnp.exp(vals_v[pl.ds(16 * c, 16)] - m)
                if c == NCHUNK - 1:
                    t = jnp.where(lane < tail, t, 0.0)
                sumv = sumv + t
            s = _lane_reduce(sumv, lax.add, lane)
            logz = _log_splat(s) + m

            for c in range(NCHUNK):
                outbuf_v[il, pl.ds(16 * c, 16)] = (
                    vals_v[pl.ds(16 * c, 16)] - logz)
            return carry

        lax.fori_loop(0, rows_per_w, row_body, 0)
        pltpu.sync_copy(outbuf_v, out_hbm.at[pl.ds(base, rows_per_w)])

    return sc_call


_SC_CALL = _make_sc_call()


def kernel(logits, word2label):
    flat = logits.reshape(ROWS * VOCAB)
    w = word2label.astype(jnp.int32)
    w = jnp.concatenate([w, jnp.zeros((LPAD - NLAB,), jnp.int32)])
    out = _SC_CALL(flat, w)
    return out[:, :NLAB]


# speedup vs baseline: 1.0164x; 1.0164x over previous
import functools

import jax
import jax.numpy as jnp
from jax import lax
from jax.experimental import pallas as pl
from jax.experimental.pallas import tpu as pltpu
from jax.experimental.pallas import tpu_sc as plsc

ROWS = 1024
VOCAB = 100000
NLAB = 100
LPAD = 112
TPAD = 128
NCHUNK = LPAD // 16
TAIL = NLAB - 16 * (NCHUNK - 1)
LN2 = 0.6931471805599453


def _lane_reduce(v, op, lane):
    dnums = lax.GatherDimensionNumbers(
        offset_dims=(), collapsed_slice_dims=(0,), start_index_map=(0,))
    for shift in (1, 2, 4, 8):
        idx = lax.bitwise_and(lane + shift, 15)
        rot = lax.gather(v, idx.reshape(16, 1), dnums, (1,),
                         mode=lax.GatherScatterMode.PROMISE_IN_BOUNDS)
        v = op(v, rot)
    return v


def _log_lanes(sv):
    bits = lax.bitcast_convert_type(sv, jnp.int32)
    e = lax.shift_right_logical(bits, 23) - 127
    mb = lax.bitwise_or(lax.bitwise_and(bits, 0x007FFFFF), 0x3F800000)
    mf = lax.bitcast_convert_type(mb, jnp.float32)
    big = mf > 1.4142135
    mf = jnp.where(big, mf * 0.5, mf)
    e = jnp.where(big, e + 1, e)
    t = (mf - 1.0) / (mf + 1.0)
    t2 = t * t
    poly = 1.0 + t2 * (
        0.3333333333 + t2 * (0.2 + t2 * (0.1428571429 + t2 * 0.1111111111)))
    return 2.0 * t * poly + e.astype(jnp.float32) * LN2


def _make_sc_call():
    info = plsc.get_sparse_core_info()
    nw = info.num_cores * info.num_subcores
    rows_per_w = ROWS // nw
    mesh = plsc.VectorSubcoreMesh(core_axis_name="c", subcore_axis_name="s")

    @functools.partial(
        pl.kernel,
        mesh=mesh,
        out_type=jax.ShapeDtypeStruct((ROWS, LPAD), jnp.float32),
        scratch_types=[
            pltpu.VMEM((LPAD,), jnp.int32),
            pltpu.VMEM((rows_per_w, TPAD), jnp.int32),
            pltpu.VMEM((rows_per_w, TPAD), jnp.float32),
            pltpu.VMEM((rows_per_w, LPAD), jnp.float32),
            pltpu.SemaphoreType.DMA,
        ],
    )
    def sc_call(flat_hbm, w_hbm, out_hbm,
                w_v, gidx_v, vals_v, outbuf_v, sem):
        wid = lax.axis_index("s") * info.num_cores + lax.axis_index("c")
        base = wid * rows_per_w
        lane = lax.iota(jnp.int32, 16)
        tailmask = lane < TAIL

        pltpu.sync_copy(w_hbm, w_v)
        wv = [w_v[pl.ds(16 * c, 16)] for c in range(NCHUNK)]

        for r in range(rows_per_w):
            rowoff = (base + r) * VOCAB
            for c in range(NCHUNK):
                gidx_v[r, pl.ds(16 * c, 16)] = wv[c] + rowoff
            gidx_v[r, pl.ds(LPAD, TPAD - LPAD)] = jnp.full(
                (TPAD - LPAD,), 0, jnp.int32) + rowoff

        copies = [
            pltpu.async_copy(flat_hbm.at[gidx_v.at[r]], vals_v.at[r], sem)
            for r in range(rows_per_w)
        ]

        for r in range(rows_per_w):
            copies[r].wait()
            v = [vals_v[r, pl.ds(16 * c, 16)] for c in range(NCHUNK)]

            maxv = jnp.where(tailmask, v[NCHUNK - 1], -3.4e38)
            for c in range(NCHUNK - 1):
                maxv = jnp.maximum(maxv, v[c])
            m = _lane_reduce(maxv, jnp.maximum, lane)

            sumv = jnp.where(tailmask, jnp.exp(v[NCHUNK - 1] - m), 0.0)
            for c in range(NCHUNK - 1):
                sumv = sumv + jnp.exp(v[c] - m)
            s = _lane_reduce(sumv, lax.add, lane)
            logz = _log_lanes(s) + m

            for c in range(NCHUNK):
                outbuf_v[r, pl.ds(16 * c, 16)] = v[c] - logz

        pltpu.sync_copy(outbuf_v, out_hbm.at[pl.ds(base, rows_per_w)])

    return sc_call


_SC_CALL = _make_sc_call()


def kernel(logits, word2label):
    flat = logits.reshape(ROWS * VOCAB)
    w = word2label.astype(jnp.int32)
    w = jnp.concatenate([w, jnp.zeros((LPAD - NLAB,), jnp.int32)])
    out = _SC_CALL(flat, w)
    return out[:, :NLAB]


# speedup vs baseline: 27.6017x; 27.1575x over previous
import functools

import jax
import jax.numpy as jnp
from jax import lax
from jax.experimental import pallas as pl
from jax.experimental.pallas import tpu as pltpu
from jax.experimental.pallas import tpu_sc as plsc

ROWS = 1024
VOCAB = 100000
NLAB = 100
LPAD = 128
LBLK = 32
RBLK = 128
NQ = 4
NH = RBLK // 16
LN2 = 0.6931471805599453
NEG = -3.4e38


def _log_lanes(sv):
    bits = lax.bitcast_convert_type(sv, jnp.int32)
    e = lax.shift_right_logical(bits, 23) - 127
    mb = lax.bitwise_or(lax.bitwise_and(bits, 0x007FFFFF), 0x3F800000)
    mf = lax.bitcast_convert_type(mb, jnp.float32)
    big = mf > 1.4142135
    mf = jnp.where(big, mf * 0.5, mf)
    e = jnp.where(big, e + 1, e)
    t = (mf - 1.0) / (mf + 1.0)
    t2 = t * t
    poly = 1.0 + t2 * (
        0.3333333333 + t2 * (0.2 + t2 * (0.1428571429 + t2 * 0.1111111111)))
    return 2.0 * t * poly + e.astype(jnp.float32) * LN2


def _make_sc_call():
    info = plsc.get_sparse_core_info()
    mesh = plsc.VectorSubcoreMesh(core_axis_name="c", subcore_axis_name="s")

    @functools.partial(
        pl.kernel,
        mesh=mesh,
        out_type=jax.ShapeDtypeStruct((LPAD, ROWS), jnp.float32),
        scratch_types=[
            pltpu.VMEM((LPAD,), jnp.int32),
            pltpu.VMEM((LBLK, 8, 128), jnp.float32),
            pltpu.VMEM((LBLK, RBLK), jnp.float32),
            pltpu.VMEM((RBLK,), jnp.float32),
            pltpu.VMEM((RBLK,), jnp.float32),
            pltpu.VMEM((NQ, RBLK), jnp.float32),
            pltpu.VMEM((NQ, RBLK), jnp.float32),
            pltpu.VMEM_SHARED((2, 16, RBLK), jnp.float32),
            pltpu.SemaphoreType.DMA,
        ],
    )
    def sc_call(lt_hbm, w_hbm, out_hbm, w_v, tiles_v, vals_v,
                pmax_v, psum_v, pm4_v, ps4_v, stats_sp, sem_in):
        c = lax.axis_index("c")
        s = lax.axis_index("s")
        b = c * 4 + s // 4
        q = s % 4
        row0 = pl.multiple_of(b * RBLK, RBLK)
        lab0 = q * LBLK

        pltpu.sync_copy(w_hbm, w_v)
        wj = []
        for cc in range(LBLK // 16):
            wc = w_v[pl.ds(lab0 + 16 * cc, 16)]
            for k in range(16):
                wj.append(wc[k])

        copies = []
        for j in range(LBLK):
            d0 = pl.multiple_of(lax.bitwise_and(wj[j], -8), 8)
            copies.append(pltpu.async_copy(
                lt_hbm.at[pl.ds(d0, 8), pl.ds(row0, RBLK)],
                tiles_v.at[j], sem_in))
        for cp in copies:
            cp.wait()

        maxacc = [jnp.full((16,), NEG, jnp.float32) for _ in range(NH)]
        for j in range(LBLK):
            dj = lax.bitwise_and(wj[j], 7)
            valid = (lab0 + j) < NLAB
            for h in range(NH):
                xl = tiles_v[j, dj, pl.ds(16 * h, 16)]
                vals_v[j, pl.ds(16 * h, 16)] = xl
                maxacc[h] = jnp.maximum(maxacc[h], jnp.where(valid, xl, NEG))
        for h in range(NH):
            pmax_v[pl.ds(16 * h, 16)] = maxacc[h]

        sumacc = [jnp.zeros((16,), jnp.float32) for _ in range(NH)]
        for j in range(LBLK):
            valid = (lab0 + j) < NLAB
            for h in range(NH):
                e = jnp.exp(vals_v[j, pl.ds(16 * h, 16)] - maxacc[h])
                sumacc[h] = sumacc[h] + jnp.where(valid, e, 0.0)
        for h in range(NH):
            psum_v[pl.ds(16 * h, 16)] = sumacc[h]

        pltpu.sync_copy(pmax_v, stats_sp.at[0, s])
        pltpu.sync_copy(psum_v, stats_sp.at[1, s])
        plsc.subcore_barrier()
        s0 = (s // 4) * 4
        pltpu.sync_copy(stats_sp.at[0, pl.ds(s0, NQ)], pm4_v)
        pltpu.sync_copy(stats_sp.at[1, pl.ds(s0, NQ)], ps4_v)

        logz = []
        for h in range(NH):
            pm = [pm4_v[r, pl.ds(16 * h, 16)] for r in range(NQ)]
            m = pm[0]
            for r in range(1, NQ):
                m = jnp.maximum(m, pm[r])
            ssum = jnp.zeros((16,), jnp.float32)
            for r in range(NQ):
                ssum = ssum + ps4_v[r, pl.ds(16 * h, 16)] * jnp.exp(pm[r] - m)
            logz.append(_log_lanes(ssum) + m)

        for j in range(LBLK):
            for h in range(NH):
                vals_v[j, pl.ds(16 * h, 16)] = (
                    vals_v[j, pl.ds(16 * h, 16)] - logz[h])
        pltpu.sync_copy(
            vals_v,
            out_hbm.at[pl.ds(pl.multiple_of(lab0, LBLK), LBLK),
                       pl.ds(row0, RBLK)])

    return sc_call


_SC_CALL = _make_sc_call()


def kernel(logits, word2label):
    lt = logits.T
    w = word2label.astype(jnp.int32)
    w = jnp.concatenate([w, jnp.zeros((LPAD - NLAB,), jnp.int32)])
    out_cm = _SC_CALL(lt, w)
    return out_cm[:NLAB].T


# speedup vs baseline: 30.3443x; 1.0994x over previous
import functools

import jax
import jax.numpy as jnp
from jax import lax
from jax.experimental import pallas as pl
from jax.experimental.pallas import tpu as pltpu
from jax.experimental.pallas import tpu_sc as plsc

ROWS = 1024
VOCAB = 100000
NLAB = 100
LPAD = 128
LBLK = 32
RBLK = 128
NQ = 4
NH = RBLK // 16
LN2 = 0.6931471805599453
NEG = -3.4e38


def _log_lanes(sv):
    bits = lax.bitcast_convert_type(sv, jnp.int32)
    e = lax.shift_right_logical(bits, 23) - 127
    mb = lax.bitwise_or(lax.bitwise_and(bits, 0x007FFFFF), 0x3F800000)
    mf = lax.bitcast_convert_type(mb, jnp.float32)
    big = mf > 1.4142135
    mf = jnp.where(big, mf * 0.5, mf)
    e = jnp.where(big, e + 1, e)
    t = (mf - 1.0) / (mf + 1.0)
    t2 = t * t
    poly = 1.0 + t2 * (
        0.3333333333 + t2 * (0.2 + t2 * (0.1428571429 + t2 * 0.1111111111)))
    return 2.0 * t * poly + e.astype(jnp.float32) * LN2


def _make_sc_call():
    info = plsc.get_sparse_core_info()
    mesh = plsc.VectorSubcoreMesh(core_axis_name="c", subcore_axis_name="s")

    @functools.partial(
        pl.kernel,
        mesh=mesh,
        out_type=jax.ShapeDtypeStruct((LPAD, ROWS), jnp.float32),
        scratch_types=[
            pltpu.VMEM((LPAD,), jnp.int32),
            pltpu.VMEM((LBLK, RBLK), jnp.float32),
            pltpu.VMEM((RBLK,), jnp.float32),
            pltpu.VMEM((RBLK,), jnp.float32),
            pltpu.VMEM((NQ, RBLK), jnp.float32),
            pltpu.VMEM((NQ, RBLK), jnp.float32),
            pltpu.VMEM_SHARED((2, 16, RBLK), jnp.float32),
            pltpu.SemaphoreType.DMA,
        ],
    )
    def sc_call(lt_hbm, w_hbm, out_hbm, w_v, vals_v,
                pmax_v, psum_v, pm4_v, ps4_v, stats_sp, sem_in):
        c = lax.axis_index("c")
        s = lax.axis_index("s")
        b = c * 4 + s // 4
        q = s % 4
        row0 = pl.multiple_of(b * RBLK, RBLK)
        lab0 = q * LBLK

        pltpu.sync_copy(w_hbm, w_v)
        wj = []
        for cc in range(LBLK // 16):
            wc = w_v[pl.ds(lab0 + 16 * cc, 16)]
            for k in range(16):
                wj.append(wc[k])

        copies = []
        for j in range(LBLK):
            copies.append(pltpu.async_copy(
                lt_hbm.at[wj[j], pl.ds(row0, RBLK)],
                vals_v.at[j], sem_in))
        for cp in copies:
            cp.wait()

        maxacc = [jnp.full((16,), NEG, jnp.float32) for _ in range(NH)]
        for j in range(LBLK):
            valid = (lab0 + j) < NLAB
            for h in range(NH):
                xl = vals_v[j, pl.ds(16 * h, 16)]
                maxacc[h] = jnp.maximum(maxacc[h], jnp.where(valid, xl, NEG))
        for h in range(NH):
            pmax_v[pl.ds(16 * h, 16)] = maxacc[h]

        sumacc = [jnp.zeros((16,), jnp.float32) for _ in range(NH)]
        for j in range(LBLK):
            valid = (lab0 + j) < NLAB
            for h in range(NH):
                e = jnp.exp(vals_v[j, pl.ds(16 * h, 16)] - maxacc[h])
                sumacc[h] = sumacc[h] + jnp.where(valid, e, 0.0)
        for h in range(NH):
            psum_v[pl.ds(16 * h, 16)] = sumacc[h]

        pltpu.sync_copy(pmax_v, stats_sp.at[0, s])
        pltpu.sync_copy(psum_v, stats_sp.at[1, s])
        plsc.subcore_barrier()
        s0 = (s // 4) * 4
        pltpu.sync_copy(stats_sp.at[0, pl.ds(s0, NQ)], pm4_v)
        pltpu.sync_copy(stats_sp.at[1, pl.ds(s0, NQ)], ps4_v)

        logz = []
        for h in range(NH):
            pm = [pm4_v[r, pl.ds(16 * h, 16)] for r in range(NQ)]
            m = pm[0]
            for r in range(1, NQ):
                m = jnp.maximum(m, pm[r])
            ssum = jnp.zeros((16,), jnp.float32)
            for r in range(NQ):
                ssum = ssum + ps4_v[r, pl.ds(16 * h, 16)] * jnp.exp(pm[r] - m)
            logz.append(_log_lanes(ssum) + m)

        for j in range(LBLK):
            for h in range(NH):
                vals_v[j, pl.ds(16 * h, 16)] = (
                    vals_v[j, pl.ds(16 * h, 16)] - logz[h])
        pltpu.sync_copy(
            vals_v,
            out_hbm.at[pl.ds(pl.multiple_of(lab0, LBLK), LBLK),
                       pl.ds(row0, RBLK)])

    return sc_call


_SC_CALL = _make_sc_call()


def kernel(logits, word2label):
    lt = logits.T
    w = word2label.astype(jnp.int32)
    w = jnp.concatenate([w, jnp.zeros((LPAD - NLAB,), jnp.int32)])
    out_cm = _SC_CALL(lt, w)
    return out_cm[:NLAB].T


# speedup vs baseline: 30.8093x; 1.0153x over previous
import functools

import jax
import jax.numpy as jnp
from jax import lax
from jax.experimental import pallas as pl
from jax.experimental.pallas import tpu as pltpu
from jax.experimental.pallas import tpu_sc as plsc

ROWS = 1024
VOCAB = 100000
NLAB = 100
LPAD = 128
LBLK = 32
RBLK = 128
NQ = 4
NH = RBLK // 16
LN2 = 0.6931471805599453
NEG = -3.4e38


def _log_lanes(sv):
    bits = lax.bitcast_convert_type(sv, jnp.int32)
    e = lax.shift_right_logical(bits, 23) - 127
    mb = lax.bitwise_or(lax.bitwise_and(bits, 0x007FFFFF), 0x3F800000)
    mf = lax.bitcast_convert_type(mb, jnp.float32)
    big = mf > 1.4142135
    mf = jnp.where(big, mf * 0.5, mf)
    e = jnp.where(big, e + 1, e)
    t = (mf - 1.0) / (mf + 1.0)
    t2 = t * t
    poly = 1.0 + t2 * (
        0.3333333333 + t2 * (0.2 + t2 * (0.1428571429 + t2 * 0.1111111111)))
    return 2.0 * t * poly + e.astype(jnp.float32) * LN2


def _make_sc_call():
    info = plsc.get_sparse_core_info()
    mesh = plsc.VectorSubcoreMesh(core_axis_name="c", subcore_axis_name="s")

    @functools.partial(
        pl.kernel,
        mesh=mesh,
        out_type=jax.ShapeDtypeStruct((LPAD, ROWS), jnp.float32),
        scratch_types=[
            pltpu.VMEM((LPAD,), jnp.int32),
            pltpu.VMEM((LBLK, RBLK), jnp.float32),
            pltpu.VMEM((2, RBLK), jnp.float32),
            pltpu.VMEM((NQ, 2, RBLK), jnp.float32),
            pltpu.VMEM_SHARED((16, 2, RBLK), jnp.float32),
            pltpu.SemaphoreType.DMA,
        ],
    )
    def sc_call(lt_hbm, w_hbm, out_hbm, w_v, vals_v,
                pstat_v, peers_v, stats_sp, sem_in):
        c = lax.axis_index("c")
        s = lax.axis_index("s")
        b = c * 4 + s // 4
        q = s % 4
        row0 = pl.multiple_of(b * RBLK, RBLK)
        lab0 = q * LBLK

        pltpu.sync_copy(w_hbm, w_v.at[pl.ds(0, NLAB)])
        wj = []
        for cc in range(LBLK // 16):
            wc = w_v[pl.ds(lab0 + 16 * cc, 16)]
            for k in range(16):
                slot = lab0 + 16 * cc + k
                wj.append(jnp.where(slot < NLAB, wc[k], 0))

        copies = []
        for j in range(LBLK):
            copies.append(pltpu.async_copy(
                lt_hbm.at[wj[j], pl.ds(row0, RBLK)],
                vals_v.at[j], sem_in))

        maxacc = [jnp.full((16,), NEG, jnp.float32) for _ in range(NH)]
        for j in range(LBLK):
            copies[j].wait()
            valid = (lab0 + j) < NLAB
            for h in range(NH):
                xl = vals_v[j, pl.ds(16 * h, 16)]
                maxacc[h] = jnp.maximum(maxacc[h], jnp.where(valid, xl, NEG))
        for h in range(NH):
            pstat_v[0, pl.ds(16 * h, 16)] = maxacc[h]

        sumacc = [jnp.zeros((16,), jnp.float32) for _ in range(NH)]
        for j in range(LBLK):
            valid = (lab0 + j) < NLAB
            for h in range(NH):
                e = jnp.exp(vals_v[j, pl.ds(16 * h, 16)] - maxacc[h])
                sumacc[h] = sumacc[h] + jnp.where(valid, e, 0.0)
        for h in range(NH):
            pstat_v[1, pl.ds(16 * h, 16)] = sumacc[h]

        pltpu.sync_copy(pstat_v, stats_sp.at[s])
        plsc.subcore_barrier()
        s0 = (s // 4) * 4
        pltpu.sync_copy(stats_sp.at[pl.ds(s0, NQ)], peers_v)

        logz = []
        for h in range(NH):
            pm = [peers_v[r, 0, pl.ds(16 * h, 16)] for r in range(NQ)]
            m = pm[0]
            for r in range(1, NQ):
                m = jnp.maximum(m, pm[r])
            ssum = jnp.zeros((16,), jnp.float32)
            for r in range(NQ):
                ssum = ssum + (peers_v[r, 1, pl.ds(16 * h, 16)]
                               * jnp.exp(pm[r] - m))
            logz.append(_log_lanes(ssum) + m)

        for j in range(LBLK):
            for h in range(NH):
                vals_v[j, pl.ds(16 * h, 16)] = (
                    vals_v[j, pl.ds(16 * h, 16)] - logz[h])
        pltpu.sync_copy(
            vals_v,
            out_hbm.at[pl.ds(pl.multiple_of(lab0, LBLK), LBLK),
                       pl.ds(row0, RBLK)])

    return sc_call


_SC_CALL = _make_sc_call()


def kernel(logits, word2label):
    lt = logits.T
    w = word2label.astype(jnp.int32)
    out_cm = _SC_CALL(lt, w)
    return out_cm[:NLAB].T


# speedup vs baseline: 32.1219x; 1.0426x over previous
import functools

import jax
import jax.numpy as jnp
from jax import lax
from jax.experimental import pallas as pl
from jax.experimental.pallas import tpu as pltpu
from jax.experimental.pallas import tpu_sc as plsc

ROWS = 1024
VOCAB = 100000
NLAB = 100
LPAD = 128
LBLK = 32
RBLK = 128
NQ = 4
NH = RBLK // 16
LN2 = 0.6931471805599453
NEG = -3.4e38


def _log_lanes(sv):
    bits = lax.bitcast_convert_type(sv, jnp.int32)
    e = lax.shift_right_logical(bits, 23) - 127
    mb = lax.bitwise_or(lax.bitwise_and(bits, 0x007FFFFF), 0x3F800000)
    mf = lax.bitcast_convert_type(mb, jnp.float32)
    big = mf > 1.4142135
    mf = jnp.where(big, mf * 0.5, mf)
    e = jnp.where(big, e + 1, e)
    t = (mf - 1.0) / (mf + 1.0)
    t2 = t * t
    poly = 1.0 + t2 * (
        0.3333333333 + t2 * (0.2 + t2 * (0.1428571429 + t2 * 0.1111111111)))
    return 2.0 * t * poly + e.astype(jnp.float32) * LN2


def _make_sc_call():
    info = plsc.get_sparse_core_info()
    mesh = plsc.VectorSubcoreMesh(core_axis_name="c", subcore_axis_name="s")

    @functools.partial(
        pl.kernel,
        mesh=mesh,
        out_type=jax.ShapeDtypeStruct((LPAD, ROWS), jnp.float32),
        scratch_types=[
            pltpu.VMEM((LPAD,), jnp.int32),
            pltpu.VMEM((LBLK, RBLK), jnp.float32),
            pltpu.VMEM((2, RBLK), jnp.float32),
            pltpu.VMEM((NQ, 2, RBLK), jnp.float32),
            pltpu.VMEM_SHARED((16, 2, RBLK), jnp.float32),
            pltpu.SemaphoreType.DMA,
        ],
    )
    def sc_call(lt_hbm, w_hbm, out_hbm, w_v, vals_v,
                pstat_v, peers_v, stats_sp, sem_in):
        c = lax.axis_index("c")
        s = lax.axis_index("s")
        b = c * 4 + s // 4
        q = s % 4
        row0 = pl.multiple_of(b * RBLK, RBLK)
        lab0 = q * LBLK

        pltpu.sync_copy(w_hbm, w_v.at[pl.ds(0, NLAB)])
        wj = []
        for cc in range(LBLK // 16):
            wc = w_v[pl.ds(lab0 + 16 * cc, 16)]
            for k in range(16):
                slot = lab0 + 16 * cc + k
                wj.append(jnp.where(slot < NLAB, wc[k], 0))

        copies = []
        for j in range(LBLK):
            copies.append(pltpu.async_copy(
                lt_hbm.at[wj[j], pl.ds(row0, RBLK)],
                vals_v.at[j], sem_in))

        maxacc = [jnp.full((16,), NEG, jnp.float32) for _ in range(NH)]
        for j in range(LBLK):
            copies[j].wait()
            valid = (lab0 + j) < NLAB
            for h in range(NH):
                xl = vals_v[j, pl.ds(16 * h, 16)]
                maxacc[h] = jnp.maximum(maxacc[h], jnp.where(valid, xl, NEG))
        for h in range(NH):
            pstat_v[0, pl.ds(16 * h, 16)] = maxacc[h]

        def p2_body(j, sumacc):
            valid = (lab0 + j) < NLAB
            out = []
            for h in range(NH):
                e = jnp.exp(vals_v[j, pl.ds(16 * h, 16)] - maxacc[h])
                out.append(sumacc[h] + jnp.where(valid, e, 0.0))
            return tuple(out)

        sumacc = lax.fori_loop(
            0, LBLK, p2_body,
            tuple(jnp.zeros((16,), jnp.float32) for _ in range(NH)))
        for h in range(NH):
            pstat_v[1, pl.ds(16 * h, 16)] = sumacc[h]

        pltpu.sync_copy(pstat_v, stats_sp.at[s])
        plsc.subcore_barrier()
        s0 = (s // 4) * 4
        pltpu.sync_copy(stats_sp.at[pl.ds(s0, NQ)], peers_v)

        logz = []
        for h in range(NH):
            pm = [peers_v[r, 0, pl.ds(16 * h, 16)] for r in range(NQ)]
            m = pm[0]
            for r in range(1, NQ):
                m = jnp.maximum(m, pm[r])
            ssum = jnp.zeros((16,), jnp.float32)
            for r in range(NQ):
                ssum = ssum + (peers_v[r, 1, pl.ds(16 * h, 16)]
                               * jnp.exp(pm[r] - m))
            logz.append(_log_lanes(ssum) + m)

        def p3_body(j, carry):
            for h in range(NH):
                vals_v[j, pl.ds(16 * h, 16)] = (
                    vals_v[j, pl.ds(16 * h, 16)] - logz[h])
            return carry

        lax.fori_loop(0, LBLK, p3_body, 0)
        pltpu.sync_copy(
            vals_v,
            out_hbm.at[pl.ds(pl.multiple_of(lab0, LBLK), LBLK),
                       pl.ds(row0, RBLK)])

    return sc_call


_SC_CALL = _make_sc_call()


def kernel(logits, word2label):
    lt = logits.T
    w = word2label.astype(jnp.int32)
    out_cm = _SC_CALL(lt, w)
    return out_cm[:NLAB].T


# speedup vs baseline: 33.6101x; 1.0463x over previous
import functools

import jax
import jax.numpy as jnp
from jax import lax
from jax.experimental import pallas as pl
from jax.experimental.pallas import tpu as pltpu
from jax.experimental.pallas import tpu_sc as plsc

ROWS = 1024
VOCAB = 100000
NLAB = 100
LPAD = 128
LBLK = 32
RBLK = 128
NQ = 4
NH = RBLK // 16
LN2 = 0.6931471805599453
NEG = -3.4e38


def _log_lanes(sv):
    bits = lax.bitcast_convert_type(sv, jnp.int32)
    e = lax.shift_right_logical(bits, 23) - 127
    mb = lax.bitwise_or(lax.bitwise_and(bits, 0x007FFFFF), 0x3F800000)
    mf = lax.bitcast_convert_type(mb, jnp.float32)
    big = mf > 1.4142135
    mf = jnp.where(big, mf * 0.5, mf)
    e = jnp.where(big, e + 1, e)
    t = (mf - 1.0) / (mf + 1.0)
    t2 = t * t
    poly = 1.0 + t2 * (
        0.3333333333 + t2 * (0.2 + t2 * (0.1428571429 + t2 * 0.1111111111)))
    return 2.0 * t * poly + e.astype(jnp.float32) * LN2


def _make_sc_call():
    info = plsc.get_sparse_core_info()
    mesh = plsc.VectorSubcoreMesh(core_axis_name="c", subcore_axis_name="s")

    @functools.partial(
        pl.kernel,
        mesh=mesh,
        out_type=jax.ShapeDtypeStruct((LPAD, ROWS), jnp.float32),
        scratch_types=[
            pltpu.VMEM((LPAD,), jnp.int32),
            pltpu.VMEM((LBLK, RBLK), jnp.float32),
            pltpu.VMEM((2, RBLK), jnp.float32),
            pltpu.VMEM((NQ, 2, RBLK), jnp.float32),
            pltpu.VMEM_SHARED((16, 2, RBLK), jnp.float32),
            pltpu.SemaphoreType.DMA,
        ],
    )
    def sc_call(lt_hbm, w_hbm, out_hbm, w_v, vals_v,
                pstat_v, peers_v, stats_sp, sem_in):
        c = lax.axis_index("c")
        s = lax.axis_index("s")
        b = c * 4 + s // 4
        q = s % 4
        row0 = pl.multiple_of(b * RBLK, RBLK)
        lab0 = q * LBLK

        pltpu.sync_copy(w_hbm, w_v.at[pl.ds(0, NLAB)])
        wj = []
        for cc in range(LBLK // 16):
            wc = w_v[pl.ds(lab0 + 16 * cc, 16)]
            for k in range(16):
                slot = lab0 + 16 * cc + k
                wj.append(jnp.where(slot < NLAB, wc[k], 0))

        for j in range(LBLK):
            pltpu.async_copy(
                lt_hbm.at[wj[j], pl.ds(row0, RBLK)],
                vals_v.at[j], sem_in)
        pltpu.make_async_copy(
            lt_hbm.at[pl.ds(0, LBLK), pl.ds(0, RBLK)], vals_v, sem_in).wait()

        def p1_body(j, maxacc):
            valid = (lab0 + j) < NLAB
            out = []
            for h in range(NH):
                xl = vals_v[j, pl.ds(16 * h, 16)]
                out.append(jnp.maximum(maxacc[h], jnp.where(valid, xl, NEG)))
            return tuple(out)

        maxacc = lax.fori_loop(
            0, LBLK, p1_body,
            tuple(jnp.full((16,), NEG, jnp.float32) for _ in range(NH)))
        for h in range(NH):
            pstat_v[0, pl.ds(16 * h, 16)] = maxacc[h]

        def p2_body(j, sumacc):
            valid = (lab0 + j) < NLAB
            out = []
            for h in range(NH):
                e = jnp.exp(vals_v[j, pl.ds(16 * h, 16)] - maxacc[h])
                out.append(sumacc[h] + jnp.where(valid, e, 0.0))
            return tuple(out)

        sumacc = lax.fori_loop(
            0, LBLK, p2_body,
            tuple(jnp.zeros((16,), jnp.float32) for _ in range(NH)))
        for h in range(NH):
            pstat_v[1, pl.ds(16 * h, 16)] = sumacc[h]

        pltpu.sync_copy(pstat_v, stats_sp.at[s])
        plsc.subcore_barrier()
        s0 = (s // 4) * 4
        pltpu.sync_copy(stats_sp.at[pl.ds(s0, NQ)], peers_v)

        def comb_body(h, carry):
            pm = [peers_v[r, 0, pl.ds(16 * h, 16)] for r in range(NQ)]
            m = pm[0]
            for r in range(1, NQ):
                m = jnp.maximum(m, pm[r])
            ssum = jnp.zeros((16,), jnp.float32)
            for r in range(NQ):
                ssum = ssum + (peers_v[r, 1, pl.ds(16 * h, 16)]
                               * jnp.exp(pm[r] - m))
            pstat_v[0, pl.ds(16 * h, 16)] = _log_lanes(ssum) + m
            return carry

        lax.fori_loop(0, NH, comb_body, 0)
        logz = [pstat_v[0, pl.ds(16 * h, 16)] for h in range(NH)]

        def p3_body(j, carry):
            for h in range(NH):
                vals_v[j, pl.ds(16 * h, 16)] = (
                    vals_v[j, pl.ds(16 * h, 16)] - logz[h])
            return carry

        lax.fori_loop(0, LBLK, p3_body, 0)
        pltpu.sync_copy(
            vals_v,
            out_hbm.at[pl.ds(pl.multiple_of(lab0, LBLK), LBLK),
                       pl.ds(row0, RBLK)])

    return sc_call


_SC_CALL = _make_sc_call()


def kernel(logits, word2label):
    lt = logits.T
    w = word2label.astype(jnp.int32)
    out_cm = _SC_CALL(lt, w)
    return out_cm[:NLAB].T
